# Initial kernel scaffold; baseline (speedup 1.0000x reference)
import functools

import jax
import jax.numpy as jnp
from jax import lax
from jax.experimental import pallas as pl
from jax.experimental.pallas import tpu as pltpu
from jax.experimental.pallas import tpu_sc as plsc

NC = 2
NS = 16
NW = NC * NS

NAP = 64
CHUNK = 128


def _sc_gather(idx2d, emb_table, prof_table):
    n_rows, d = idx2d.shape[0] * CHUNK, emb_table.shape[1]
    chunks_per_w = idx2d.shape[0] // NW
    mesh = plsc.VectorSubcoreMesh(core_axis_name="c", subcore_axis_name="s")

    @functools.partial(
        pl.kernel,
        out_type=(jax.ShapeDtypeStruct((n_rows, d), jnp.float32),
                  jax.ShapeDtypeStruct((n_rows, d), jnp.float32)),
        mesh=mesh,
        scratch_types=[
            pltpu.VMEM((chunks_per_w, CHUNK), jnp.int32),
            pltpu.VMEM((CHUNK, d), jnp.float32),
            pltpu.VMEM((CHUNK, d), jnp.float32),
            pltpu.SemaphoreType.DMA,
            pltpu.SemaphoreType.DMA,
        ],
    )
    def gather_kernel(idx_hbm, emb_hbm, prof_hbm, out_e, out_p,
                      idx_v, ebuf, pbuf, sem_e, sem_p):
        wid = lax.axis_index("s") * NC + lax.axis_index("c")
        chunk0 = wid * chunks_per_w
        pltpu.sync_copy(idx_hbm.at[pl.ds(chunk0, chunks_per_w)], idx_v)
        for j in range(chunks_per_w):
            ce = pltpu.async_copy(emb_hbm.at[idx_v.at[j]], ebuf, sem_e)
            cp = pltpu.async_copy(prof_hbm.at[idx_v.at[j]], pbuf, sem_p)
            ce.wait()
            cp.wait()
            row0 = (chunk0 + j) * CHUNK
            pltpu.sync_copy(ebuf, out_e.at[pl.ds(row0, CHUNK)])
            pltpu.sync_copy(pbuf, out_p.at[pl.ds(row0, CHUNK)])

    return gather_kernel(idx2d, emb_table, prof_table)


def _tc_dense(emb_rows, prof_rows, u_embs, i_embs, wf, bf2, wc1t, wc1b, bc1_2,
              wc2, bc2_2, block_b):
    b, d = u_embs.shape

    def body(emb_ref, prof_ref, u_ref, i_ref, wf_ref, bf_ref, wc1t_ref,
             wc1b_ref, bc1_ref, wc2_ref, bc2_ref, comb_ref, attn_ref):
        m = block_b * NAP
        x = jnp.concatenate([emb_ref[...], prof_ref[...]], axis=-1)
        h = jnp.maximum(
            jnp.dot(x, wf_ref[...], preferred_element_type=jnp.float32)
            + bf_ref[...], 0.0)
        ic = jnp.dot(i_ref[...], wc1b_ref[...],
                     preferred_element_type=jnp.float32)
        t = jnp.dot(h, wc1t_ref[...], preferred_element_type=jnp.float32)
        t3 = t.reshape(block_b, NAP, d)
        c1 = jnp.maximum(t3 + ic[:, None, :] + bc1_ref[...], 0.0)
        c2 = jnp.maximum(
            jnp.dot(c1.reshape(m, d), wc2_ref[...],
                    preferred_element_type=jnp.float32).reshape(block_b, NAP, d)
            + bc2_ref[...], 0.0)
        scores = jnp.sum(c2 * u_ref[...][:, None, :], axis=2)
        nid = lax.broadcasted_iota(jnp.int32, (block_b, NAP), 1)
        scores = jnp.where(nid < 50, scores, -jnp.inf)
        mx = jnp.max(scores, axis=1, keepdims=True)
        e = jnp.exp(scores - mx)
        attn = e / jnp.sum(e, axis=1, keepdims=True)
        comb_ref[...] = jnp.sum(c2 * attn[:, :, None], axis=1)
        attn_ref[...] = attn

    grid = (b // block_b,)
    mb = block_b * NAP
    return pl.pallas_call(
        body,
        grid=grid,
        in_specs=[
            pl.BlockSpec((mb, d), lambda i: (i, 0)),
            pl.BlockSpec((mb, d), lambda i: (i, 0)),
            pl.BlockSpec((block_b, d), lambda i: (i, 0)),
            pl.BlockSpec((block_b, d), lambda i: (i, 0)),
            pl.BlockSpec((2 * d, d), lambda i: (0, 0)),
            pl.BlockSpec((1, d), lambda i: (0, 0)),
            pl.BlockSpec((d, d), lambda i: (0, 0)),
            pl.BlockSpec((d, d), lambda i: (0, 0)),
            pl.BlockSpec((1, d), lambda i: (0, 0)),
            pl.BlockSpec((d, d), lambda i: (0, 0)),
            pl.BlockSpec((1, d), lambda i: (0, 0)),
        ],
        out_specs=[
            pl.BlockSpec((block_b, d), lambda i: (i, 0)),
            pl.BlockSpec((block_b, NAP), lambda i: (i, 0)),
        ],
        out_shape=[
            jax.ShapeDtypeStruct((b, d), jnp.float32),
            jax.ShapeDtypeStruct((b, NAP), jnp.float32),
        ],
    )(emb_rows, prof_rows, u_embs, i_embs, wf, bf2, wc1t, wc1b, bc1_2, wc2,
      bc2_2)


def kernel(users, u_embs, items, i_embs, act_users, user_embs_weight,
           user_profiles, Wf, bf, Wc1, bc1, Wc2, bc2):
    b, na = act_users.shape
    d = u_embs.shape[1]
    au = jnp.zeros((b, NAP), jnp.int32).at[:, :na].set(
        act_users.astype(jnp.int32))
    idx2d = au.reshape(b * NAP // CHUNK, CHUNK)

    emb_rows, prof_rows = _sc_gather(idx2d, user_embs_weight, user_profiles)

    comb, attn_full = _tc_dense(
        emb_rows, prof_rows, u_embs, i_embs,
        Wf, bf.reshape(1, d),
        Wc1[:d, :], Wc1[d:, :], bc1.reshape(1, d),
        Wc2, bc2.reshape(1, d),
        block_b=128)

    return comb, attn_full[:, :na, None]


# speedup vs baseline: 1.1618x; 1.1618x over previous
import functools

import jax
import jax.numpy as jnp
from jax import lax
from jax.experimental import pallas as pl
from jax.experimental.pallas import tpu as pltpu
from jax.experimental.pallas import tpu_sc as plsc

NC = 2
NS = 16
NW = NC * NS

NAP = 64
CHUNK = 128


def _sc_gather(idx2d, emb_table, prof_table):
    n_rows, d = idx2d.shape[0] * CHUNK, emb_table.shape[1]
    chunks_per_w = idx2d.shape[0] // NW
    mesh = plsc.VectorSubcoreMesh(core_axis_name="c", subcore_axis_name="s")

    @functools.partial(
        pl.kernel,
        out_type=(jax.ShapeDtypeStruct((n_rows, d), jnp.float32),
                  jax.ShapeDtypeStruct((n_rows, d), jnp.float32)),
        mesh=mesh,
        compiler_params=pltpu.CompilerParams(use_tc_tiling_on_sc=False),
        scratch_types=[
            pltpu.VMEM((chunks_per_w, CHUNK), jnp.int32),
            pltpu.VMEM((CHUNK, d), jnp.float32),
            pltpu.VMEM((CHUNK, d), jnp.float32),
            pltpu.SemaphoreType.DMA,
            pltpu.SemaphoreType.DMA,
        ],
    )
    def gather_kernel(idx_hbm, emb_hbm, prof_hbm, out_e, out_p,
                      idx_v, ebuf, pbuf, sem_e, sem_p):
        wid = lax.axis_index("s") * NC + lax.axis_index("c")
        chunk0 = wid * chunks_per_w
        pltpu.sync_copy(idx_hbm.at[pl.ds(chunk0, chunks_per_w)], idx_v)
        for j in range(chunks_per_w):
            ce = pltpu.async_copy(emb_hbm.at[idx_v.at[j]], ebuf, sem_e)
            cp = pltpu.async_copy(prof_hbm.at[idx_v.at[j]], pbuf, sem_p)
            ce.wait()
            cp.wait()
            row0 = (chunk0 + j) * CHUNK
            pltpu.sync_copy(ebuf, out_e.at[pl.ds(row0, CHUNK)])
            pltpu.sync_copy(pbuf, out_p.at[pl.ds(row0, CHUNK)])

    return gather_kernel(idx2d, emb_table, prof_table)


def _tc_dense(emb_rows, prof_rows, u_embs, i_embs, wf, bf2, wc1t, wc1b, bc1_2,
              wc2, bc2_2, block_b):
    b, d = u_embs.shape

    def body(emb_ref, prof_ref, u_ref, i_ref, wf_ref, bf_ref, wc1t_ref,
             wc1b_ref, bc1_ref, wc2_ref, bc2_ref, comb_ref, attn_ref):
        m = block_b * NAP
        x = jnp.concatenate([emb_ref[...], prof_ref[...]], axis=-1)
        h = jnp.maximum(
            jnp.dot(x, wf_ref[...], preferred_element_type=jnp.float32)
            + bf_ref[...], 0.0)
        ic = jnp.dot(i_ref[...], wc1b_ref[...],
                     preferred_element_type=jnp.float32)
        t = jnp.dot(h, wc1t_ref[...], preferred_element_type=jnp.float32)
        t3 = t.reshape(block_b, NAP, d)
        c1 = jnp.maximum(t3 + ic[:, None, :] + bc1_ref[...], 0.0)
        c2 = jnp.maximum(
            jnp.dot(c1.reshape(m, d), wc2_ref[...],
                    preferred_element_type=jnp.float32).reshape(block_b, NAP, d)
            + bc2_ref[...], 0.0)
        scores = jnp.sum(c2 * u_ref[...][:, None, :], axis=2)
        nid = lax.broadcasted_iota(jnp.int32, (block_b, NAP), 1)
        scores = jnp.where(nid < 50, scores, -jnp.inf)
        mx = jnp.max(scores, axis=1, keepdims=True)
        e = jnp.exp(scores - mx)
        attn = e / jnp.sum(e, axis=1, keepdims=True)
        comb_ref[...] = jnp.sum(c2 * attn[:, :, None], axis=1)
        attn_ref[...] = attn

    grid = (b // block_b,)
    mb = block_b * NAP
    return pl.pallas_call(
        body,
        grid=grid,
        in_specs=[
            pl.BlockSpec((mb, d), lambda i: (i, 0)),
            pl.BlockSpec((mb, d), lambda i: (i, 0)),
            pl.BlockSpec((block_b, d), lambda i: (i, 0)),
            pl.BlockSpec((block_b, d), lambda i: (i, 0)),
            pl.BlockSpec((2 * d, d), lambda i: (0, 0)),
            pl.BlockSpec((1, d), lambda i: (0, 0)),
            pl.BlockSpec((d, d), lambda i: (0, 0)),
            pl.BlockSpec((d, d), lambda i: (0, 0)),
            pl.BlockSpec((1, d), lambda i: (0, 0)),
            pl.BlockSpec((d, d), lambda i: (0, 0)),
            pl.BlockSpec((1, d), lambda i: (0, 0)),
        ],
        out_specs=[
            pl.BlockSpec((block_b, d), lambda i: (i, 0)),
            pl.BlockSpec((block_b, NAP), lambda i: (i, 0)),
        ],
        out_shape=[
            jax.ShapeDtypeStruct((b, d), jnp.float32),
            jax.ShapeDtypeStruct((b, NAP), jnp.float32),
        ],
    )(emb_rows, prof_rows, u_embs, i_embs, wf, bf2, wc1t, wc1b, bc1_2, wc2,
      bc2_2)


def kernel(users, u_embs, items, i_embs, act_users, user_embs_weight,
           user_profiles, Wf, bf, Wc1, bc1, Wc2, bc2):
    b, na = act_users.shape
    d = u_embs.shape[1]
    au = jnp.zeros((b, NAP), jnp.int32).at[:, :na].set(
        act_users.astype(jnp.int32))
    idx2d = au.reshape(b * NAP // CHUNK, CHUNK)

    emb_rows, prof_rows = _sc_gather(idx2d, user_embs_weight, user_profiles)

    comb, attn_full = _tc_dense(
        emb_rows, prof_rows, u_embs, i_embs,
        Wf, bf.reshape(1, d),
        Wc1[:d, :], Wc1[d:, :], bc1.reshape(1, d),
        Wc2, bc2.reshape(1, d),
        block_b=128)

    return comb, attn_full[:, :na, None]


# speedup vs baseline: 1.6984x; 1.4619x over previous
import functools

import jax
import jax.numpy as jnp
from jax import lax
from jax.experimental import pallas as pl
from jax.experimental.pallas import tpu as pltpu
from jax.experimental.pallas import tpu_sc as plsc

NC = 2
NS = 16
NW = NC * NS

NAP = 56
CHUNK = 128
NB = 4


def _sc_gather(idx2d, emb_table, prof_table):
    n_rows, d = idx2d.shape[0] * CHUNK, emb_table.shape[1]
    chunks_per_w = idx2d.shape[0] // NW
    mesh = plsc.VectorSubcoreMesh(core_axis_name="c", subcore_axis_name="s")

    @functools.partial(
        pl.kernel,
        out_type=(jax.ShapeDtypeStruct((n_rows, d), jnp.float32),
                  jax.ShapeDtypeStruct((n_rows, d), jnp.float32)),
        mesh=mesh,
        compiler_params=pltpu.CompilerParams(use_tc_tiling_on_sc=False),
        scratch_types=[
            pltpu.VMEM((chunks_per_w, CHUNK), jnp.int32),
            pltpu.VMEM((NB, CHUNK, d), jnp.float32),
            pltpu.VMEM((NB, CHUNK, d), jnp.float32),
            [pltpu.SemaphoreType.DMA] * NB,
            [pltpu.SemaphoreType.DMA] * NB,
            [pltpu.SemaphoreType.DMA] * NB,
            [pltpu.SemaphoreType.DMA] * NB,
        ],
    )
    def gather_kernel(idx_hbm, emb_hbm, prof_hbm, out_e, out_p,
                      idx_v, ebuf, pbuf, ges, gps, wes, wps):
        wid = lax.axis_index("s") * NC + lax.axis_index("c")
        chunk0 = wid * chunks_per_w
        pltpu.sync_copy(idx_hbm.at[pl.ds(chunk0, chunks_per_w)], idx_v)

        gd = [None] * NB
        wd = [None] * NB

        def issue_gather(j):
            s = j % NB
            gd[s] = (pltpu.async_copy(emb_hbm.at[idx_v.at[j]], ebuf.at[s],
                                      ges[s]),
                     pltpu.async_copy(prof_hbm.at[idx_v.at[j]], pbuf.at[s],
                                      gps[s]))

        def issue_write(j):
            s = j % NB
            row0 = (chunk0 + j) * CHUNK
            wd[s] = (pltpu.async_copy(ebuf.at[s], out_e.at[pl.ds(row0, CHUNK)],
                                      wes[s]),
                     pltpu.async_copy(pbuf.at[s], out_p.at[pl.ds(row0, CHUNK)],
                                      wps[s]))

        for j in range(NB):
            issue_gather(j)
        for j in range(chunks_per_w):
            s = j % NB
            gd[s][0].wait()
            gd[s][1].wait()
            issue_write(j)
            nxt = j + NB
            if nxt < chunks_per_w:
                wd[s][0].wait()
                wd[s][1].wait()
                issue_gather(nxt)
        for j in range(max(chunks_per_w - NB, 0), chunks_per_w):
            s = j % NB
            wd[s][0].wait()
            wd[s][1].wait()

    return gather_kernel(idx2d, emb_table, prof_table)


def _tc_dense(emb_rows, prof_rows, u_embs, i_embs, wf, bf2, wc1t, wc1b, bc1_2,
              wc2, bc2_2, block_b, na):
    b, d = u_embs.shape

    def body(emb_ref, prof_ref, u_ref, i_ref, wf_ref, bf_ref, wc1t_ref,
             wc1b_ref, bc1_ref, wc2_ref, bc2_ref, comb_ref, attn_ref):
        m = block_b * NAP
        x = jnp.concatenate([emb_ref[...], prof_ref[...]], axis=-1)
        h = jnp.maximum(
            jnp.dot(x, wf_ref[...], preferred_element_type=jnp.float32)
            + bf_ref[...], 0.0)
        ic = jnp.dot(i_ref[...], wc1b_ref[...],
                     preferred_element_type=jnp.float32)
        t = jnp.dot(h, wc1t_ref[...], preferred_element_type=jnp.float32)
        t3 = t.reshape(block_b, NAP, d)
        c1 = jnp.maximum(t3 + ic[:, None, :] + bc1_ref[...], 0.0)
        c2 = jnp.maximum(
            jnp.dot(c1.reshape(m, d), wc2_ref[...],
                    preferred_element_type=jnp.float32).reshape(block_b, NAP, d)
            + bc2_ref[...], 0.0)
        scores = jnp.sum(c2 * u_ref[...][:, None, :], axis=2)
        nid = lax.broadcasted_iota(jnp.int32, (block_b, NAP), 1)
        scores = jnp.where(nid < na, scores, -jnp.inf)
        mx = jnp.max(scores, axis=1, keepdims=True)
        e = jnp.exp(scores - mx)
        attn = e / jnp.sum(e, axis=1, keepdims=True)
        comb_ref[...] = jnp.sum(c2 * attn[:, :, None], axis=1)
        attn_ref[...] = attn

    grid = (b // block_b,)
    mb = block_b * NAP
    return pl.pallas_call(
        body,
        grid=grid,
        in_specs=[
            pl.BlockSpec((mb, d), lambda i: (i, 0)),
            pl.BlockSpec((mb, d), lambda i: (i, 0)),
            pl.BlockSpec((block_b, d), lambda i: (i, 0)),
            pl.BlockSpec((block_b, d), lambda i: (i, 0)),
            pl.BlockSpec((2 * d, d), lambda i: (0, 0)),
            pl.BlockSpec((1, d), lambda i: (0, 0)),
            pl.BlockSpec((d, d), lambda i: (0, 0)),
            pl.BlockSpec((d, d), lambda i: (0, 0)),
            pl.BlockSpec((1, d), lambda i: (0, 0)),
            pl.BlockSpec((d, d), lambda i: (0, 0)),
            pl.BlockSpec((1, d), lambda i: (0, 0)),
        ],
        out_specs=[
            pl.BlockSpec((block_b, d), lambda i: (i, 0)),
            pl.BlockSpec((block_b, NAP), lambda i: (i, 0)),
        ],
        out_shape=[
            jax.ShapeDtypeStruct((b, d), jnp.float32),
            jax.ShapeDtypeStruct((b, NAP), jnp.float32),
        ],
    )(emb_rows, prof_rows, u_embs, i_embs, wf, bf2, wc1t, wc1b, bc1_2, wc2,
      bc2_2)


def kernel(users, u_embs, items, i_embs, act_users, user_embs_weight,
           user_profiles, Wf, bf, Wc1, bc1, Wc2, bc2):
    b, na = act_users.shape
    d = u_embs.shape[1]
    au = jnp.zeros((b, NAP), jnp.int32).at[:, :na].set(
        act_users.astype(jnp.int32))
    idx2d = au.reshape(b * NAP // CHUNK, CHUNK)

    emb_rows, prof_rows = _sc_gather(idx2d, user_embs_weight, user_profiles)

    comb, attn_full = _tc_dense(
        emb_rows, prof_rows, u_embs, i_embs,
        Wf, bf.reshape(1, d),
        Wc1[:d, :], Wc1[d:, :], bc1.reshape(1, d),
        Wc2, bc2.reshape(1, d),
        block_b=128, na=na)

    return comb, attn_full[:, :na, None]


# speedup vs baseline: 1.9403x; 1.1424x over previous
import functools

import jax
import jax.numpy as jnp
from jax import lax
from jax.experimental import pallas as pl
from jax.experimental.pallas import tpu as pltpu
from jax.experimental.pallas import tpu_sc as plsc

NC = 2
NS = 16
NW = NC * NS

NAP = 56
CHUNK = 128
NB = 6


def _sc_gather(idx2d, emb_table, prof_table):
    n_rows, d = idx2d.shape[0] * CHUNK, emb_table.shape[1]
    chunks_per_w = idx2d.shape[0] // NW
    mesh = plsc.VectorSubcoreMesh(core_axis_name="c", subcore_axis_name="s")

    @functools.partial(
        pl.kernel,
        out_type=jax.ShapeDtypeStruct((n_rows, 2 * d), jnp.float32),
        mesh=mesh,
        compiler_params=pltpu.CompilerParams(use_tc_tiling_on_sc=False),
        scratch_types=[
            pltpu.VMEM((chunks_per_w, CHUNK), jnp.int32),
            pltpu.VMEM((NB, CHUNK, d), jnp.float32),
            pltpu.VMEM((NB, CHUNK, d), jnp.float32),
            [pltpu.SemaphoreType.DMA] * NB,
            [pltpu.SemaphoreType.DMA] * NB,
            [pltpu.SemaphoreType.DMA] * NB,
            [pltpu.SemaphoreType.DMA] * NB,
        ],
    )
    def gather_kernel(idx_hbm, emb_hbm, prof_hbm, out,
                      idx_v, ebuf, pbuf, ges, gps, wes, wps):
        wid = lax.axis_index("s") * NC + lax.axis_index("c")
        chunk0 = wid * chunks_per_w
        pltpu.sync_copy(idx_hbm.at[pl.ds(chunk0, chunks_per_w)], idx_v)

        gd = [None] * NB
        wd = [None] * NB

        def issue_gather(j):
            s = j % NB
            gd[s] = (pltpu.async_copy(emb_hbm.at[idx_v.at[j]], ebuf.at[s],
                                      ges[s]),
                     pltpu.async_copy(prof_hbm.at[idx_v.at[j]], pbuf.at[s],
                                      gps[s]))

        def issue_write(j):
            s = j % NB
            row0 = (chunk0 + j) * CHUNK
            wd[s] = (pltpu.async_copy(
                         ebuf.at[s], out.at[pl.ds(row0, CHUNK), pl.ds(0, d)],
                         wes[s]),
                     pltpu.async_copy(
                         pbuf.at[s], out.at[pl.ds(row0, CHUNK), pl.ds(d, d)],
                         wps[s]))

        for j in range(NB):
            issue_gather(j)
        for j in range(chunks_per_w):
            s = j % NB
            gd[s][0].wait()
            gd[s][1].wait()
            issue_write(j)
            nxt = j + NB
            if nxt < chunks_per_w:
                wd[s][0].wait()
                wd[s][1].wait()
                issue_gather(nxt)
        for j in range(max(chunks_per_w - NB, 0), chunks_per_w):
            s = j % NB
            wd[s][0].wait()
            wd[s][1].wait()

    return gather_kernel(idx2d, emb_table, prof_table)


def _tc_dense(x_rows, u_embs, i_embs, wf, bf2, wc1t, wc1b, bc1_2,
              wc2, bc2_2, block_b, na):
    b, d = u_embs.shape

    def body(x_ref, u_ref, i_ref, wf_ref, bf_ref, wc1t_ref,
             wc1b_ref, bc1_ref, wc2_ref, bc2_ref, comb_ref, attn_ref):
        m = block_b * NAP
        x = x_ref[...]
        h = jnp.maximum(
            jnp.dot(x, wf_ref[...], preferred_element_type=jnp.float32)
            + bf_ref[...], 0.0)
        ic = jnp.dot(i_ref[...], wc1b_ref[...],
                     preferred_element_type=jnp.float32)
        t = jnp.dot(h, wc1t_ref[...], preferred_element_type=jnp.float32)
        t3 = t.reshape(block_b, NAP, d)
        c1 = jnp.maximum(t3 + ic[:, None, :] + bc1_ref[...], 0.0)
        c2 = jnp.maximum(
            jnp.dot(c1.reshape(m, d), wc2_ref[...],
                    preferred_element_type=jnp.float32).reshape(block_b, NAP, d)
            + bc2_ref[...], 0.0)
        scores = jnp.sum(c2 * u_ref[...][:, None, :], axis=2)
        nid = lax.broadcasted_iota(jnp.int32, (block_b, NAP), 1)
        scores = jnp.where(nid < na, scores, -jnp.inf)
        mx = jnp.max(scores, axis=1, keepdims=True)
        e = jnp.exp(scores - mx)
        attn = e / jnp.sum(e, axis=1, keepdims=True)
        comb_ref[...] = jnp.sum(c2 * attn[:, :, None], axis=1)
        attn_ref[...] = attn

    grid = (b // block_b,)
    mb = block_b * NAP
    return pl.pallas_call(
        body,
        grid=grid,
        in_specs=[
            pl.BlockSpec((mb, 2 * d), lambda i: (i, 0)),
            pl.BlockSpec((block_b, d), lambda i: (i, 0)),
            pl.BlockSpec((block_b, d), lambda i: (i, 0)),
            pl.BlockSpec((2 * d, d), lambda i: (0, 0)),
            pl.BlockSpec((1, d), lambda i: (0, 0)),
            pl.BlockSpec((d, d), lambda i: (0, 0)),
            pl.BlockSpec((d, d), lambda i: (0, 0)),
            pl.BlockSpec((1, d), lambda i: (0, 0)),
            pl.BlockSpec((d, d), lambda i: (0, 0)),
            pl.BlockSpec((1, d), lambda i: (0, 0)),
        ],
        out_specs=[
            pl.BlockSpec((block_b, d), lambda i: (i, 0)),
            pl.BlockSpec((block_b, NAP), lambda i: (i, 0)),
        ],
        out_shape=[
            jax.ShapeDtypeStruct((b, d), jnp.float32),
            jax.ShapeDtypeStruct((b, NAP), jnp.float32),
        ],
    )(x_rows, u_embs, i_embs, wf, bf2, wc1t, wc1b, bc1_2, wc2,
      bc2_2)


def kernel(users, u_embs, items, i_embs, act_users, user_embs_weight,
           user_profiles, Wf, bf, Wc1, bc1, Wc2, bc2):
    b, na = act_users.shape
    d = u_embs.shape[1]
    au = jnp.zeros((b, NAP), jnp.int32).at[:, :na].set(
        act_users.astype(jnp.int32))
    idx2d = au.reshape(b * NAP // CHUNK, CHUNK)

    x_rows = _sc_gather(idx2d, user_embs_weight, user_profiles)

    comb, attn_full = _tc_dense(
        x_rows, u_embs, i_embs,
        Wf, bf.reshape(1, d),
        Wc1[:d, :], Wc1[d:, :], bc1.reshape(1, d),
        Wc2, bc2.reshape(1, d),
        block_b=128, na=na)

    return comb, attn_full[:, :na, None]
